# Initial kernel scaffold; baseline (speedup 1.0000x reference)
import jax
import jax.numpy as jnp
from jax.experimental import pallas as pl


def kernel(keypoints, scores, W0, b0, W1, b1, W2, b2, We, be, Wp, bp):
    raise NotImplementedError("write your pallas kernel here")


# speedup vs baseline: 17.2970x; 17.2970x over previous
import functools

import jax
import jax.numpy as jnp
from jax.experimental import pallas as pl

NJ = 17
SUBSETS = [[0, 5, 6, 11, 12], [7, 8, 13, 14], [9, 10, 15, 16]]
_ms = [frozenset(s) for s in SUBSETS]
NB = [sorted(_ms[0] | _ms[1]), sorted(_ms[0] | _ms[1] | _ms[2]), sorted(_ms[1] | _ms[2])]
MASKS = [sorted(s) for s in _ms]
NEG = -1e30


def _body(kx_ref, ky_ref, sc_ref, w0_ref, w1_ref, w2_ref, b0_ref, b1_ref,
          b2_ref, e1_ref, e2_ref, be_ref, wp_ref, bp_ref, out_ref):
    kx = kx_ref[...]
    ky = ky_ref[...]
    sc = sc_ref[...]
    mnx = jnp.min(kx, axis=1, keepdims=True)
    mxx = jnp.max(kx, axis=1, keepdims=True)
    mny = jnp.min(ky, axis=1, keepdims=True)
    mxy = jnp.max(ky, axis=1, keepdims=True)
    nx = (kx - mnx) / (mxx - mnx + 1e-6)
    ny = (ky - mny) / (mxy - mny + 1e-6)

    xs = [nx[:, j:j + 1] for j in range(NJ)]
    ys = [ny[:, j:j + 1] for j in range(NJ)]
    ss = [sc[:, j:j + 1] for j in range(NJ)]

    aw = e1_ref[...] - e2_ref[...]
    cw = e2_ref[...]
    be = be_ref[...]
    wts = [w0_ref, w1_ref, w2_ref]
    bs = [b0_ref, b1_ref, b2_ref]

    acc = jnp.broadcast_to(bp_ref[...], out_ref.shape)
    for lvl in range(3):
        wt = wts[lvl][...]
        bl = bs[lvl][...]
        wx, wy, ws = wt[0:1, :], wt[1:2, :], wt[2:3, :]
        f = {}
        for j in NB[lvl]:
            h = jnp.maximum(xs[j] * wx + ys[j] * wy + ss[j] * ws + bl, 0.0)
            f[j] = h * ss[j]
        ssum = functools.reduce(jnp.add, [f[j] for j in MASKS[lvl]])
        c = {k: jnp.dot(f[k], cw, preferred_element_type=jnp.float32) + be
             for k in NB[lvl]}
        a = {j: jnp.dot(f[j], aw, preferred_element_type=jnp.float32)
             for j in MASKS[lvl]}
        m1 = functools.reduce(jnp.maximum, [c[k] for k in NB[lvl]])
        first = {}
        seen = None
        for k in NB[lvl]:
            eq = c[k] == m1
            first[k] = eq if seen is None else (eq & (~seen))
            seen = eq if seen is None else (seen | eq)
        m2 = functools.reduce(
            jnp.maximum,
            [jnp.where(first[k], NEG, c[k]) for k in NB[lvl]])
        zsum = functools.reduce(jnp.add, [
            jnp.maximum(a[j] + jnp.where(first[j], m2, m1), 0.0)
            for j in MASKS[lvl]])
        inv_cnt = 1.0 / float(len(MASKS[lvl]))
        wp_s = wp_ref[128 * lvl:128 * lvl + 64, :]
        wp_z = wp_ref[128 * lvl + 64:128 * lvl + 128, :]
        acc = acc + jnp.dot(ssum, wp_s, preferred_element_type=jnp.float32)
        acc = acc + jnp.dot(zsum * inv_cnt, wp_z,
                            preferred_element_type=jnp.float32)
    out_ref[...] = acc


def _encode(kx, ky, sc, w0t, w1t, w2t, b0, b1, b2, e1, e2, be, wpt, bp,
            block_n):
    n = kx.shape[0]
    grid = (n // block_n,)
    data_spec = pl.BlockSpec((block_n, NJ), lambda i: (i, 0))

    def rep(shape):
        return pl.BlockSpec(shape, lambda i: tuple(0 for _ in shape))

    return pl.pallas_call(
        _body,
        grid=grid,
        in_specs=[
            data_spec, data_spec, data_spec,
            rep((3, 64)), rep((3, 64)), rep((3, 64)),
            rep((1, 64)), rep((1, 64)), rep((1, 64)),
            rep((64, 64)), rep((64, 64)), rep((1, 64)),
            rep((384, 128)), rep((1, 128)),
        ],
        out_specs=pl.BlockSpec((block_n, 128), lambda i: (i, 0)),
        out_shape=jax.ShapeDtypeStruct((n, 128), jnp.float32),
    )(kx, ky, sc, w0t, w1t, w2t, b0, b1, b2, e1, e2, be, wpt, bp)


def kernel(keypoints, scores, W0, b0, W1, b1, W2, b2, We, be, Wp, bp):
    kx = keypoints[:, :, 0]
    ky = keypoints[:, :, 1]
    return _encode(
        kx, ky, scores,
        W0.T, W1.T, W2.T,
        b0[None, :], b1[None, :], b2[None, :],
        We[:, :64].T, We[:, 64:].T, be[None, :],
        Wp.T, bp[None, :],
        block_n=256)


# speedup vs baseline: 21.8233x; 1.2617x over previous
import functools

import jax
import jax.numpy as jnp
from jax.experimental import pallas as pl

NJ = 17
SUBSETS = [[0, 5, 6, 11, 12], [7, 8, 13, 14], [9, 10, 15, 16]]
_ms = [frozenset(s) for s in SUBSETS]
NB = [sorted(_ms[0] | _ms[1]), sorted(_ms[0] | _ms[1] | _ms[2]), sorted(_ms[1] | _ms[2])]
MASKS = [sorted(s) for s in _ms]
NEG = -1e30


def _body(kx_ref, ky_ref, sc_ref, w0_ref, w1_ref, w2_ref, b0_ref, b1_ref,
          b2_ref, e1_ref, e2_ref, be_ref, wp_ref, bp_ref, out_ref):
    kx = kx_ref[...]
    ky = ky_ref[...]
    sc = sc_ref[...]
    mnx = jnp.min(kx, axis=1, keepdims=True)
    mxx = jnp.max(kx, axis=1, keepdims=True)
    mny = jnp.min(ky, axis=1, keepdims=True)
    mxy = jnp.max(ky, axis=1, keepdims=True)
    nx = (kx - mnx) / (mxx - mnx + 1e-6)
    ny = (ky - mny) / (mxy - mny + 1e-6)

    xs = [nx[:, j:j + 1] for j in range(NJ)]
    ys = [ny[:, j:j + 1] for j in range(NJ)]
    ss = [sc[:, j:j + 1] for j in range(NJ)]

    aw = e1_ref[...] - e2_ref[...]
    cw = e2_ref[...]
    be = be_ref[...]
    wts = [w0_ref, w1_ref, w2_ref]
    bs = [b0_ref, b1_ref, b2_ref]

    acc = jnp.broadcast_to(bp_ref[...], out_ref.shape)
    for lvl in range(3):
        wt = wts[lvl][...]
        bl = bs[lvl][...]
        wx, wy, ws = wt[0:1, :], wt[1:2, :], wt[2:3, :]
        f = {}
        for j in NB[lvl]:
            h = jnp.maximum(xs[j] * wx + ys[j] * wy + ss[j] * ws + bl, 0.0)
            f[j] = h * ss[j]
        ssum = functools.reduce(jnp.add, [f[j] for j in MASKS[lvl]])
        c = {k: jnp.dot(f[k], cw, preferred_element_type=jnp.float32) + be
             for k in NB[lvl]}
        a = {j: jnp.dot(f[j], aw, preferred_element_type=jnp.float32)
             for j in MASKS[lvl]}
        ks = NB[lvl]
        m1 = jnp.maximum(c[ks[0]], c[ks[1]])
        m2 = jnp.minimum(c[ks[0]], c[ks[1]])
        for k in ks[2:]:
            m2 = jnp.maximum(m2, jnp.minimum(m1, c[k]))
            m1 = jnp.maximum(m1, c[k])
        zsum = functools.reduce(jnp.add, [
            jnp.maximum(a[j] + jnp.where(c[j] == m1, m2, m1), 0.0)
            for j in MASKS[lvl]])
        inv_cnt = 1.0 / float(len(MASKS[lvl]))
        wp_s = wp_ref[128 * lvl:128 * lvl + 64, :]
        wp_z = wp_ref[128 * lvl + 64:128 * lvl + 128, :]
        acc = acc + jnp.dot(ssum, wp_s, preferred_element_type=jnp.float32)
        acc = acc + jnp.dot(zsum * inv_cnt, wp_z,
                            preferred_element_type=jnp.float32)
    out_ref[...] = acc


def _encode(kx, ky, sc, w0t, w1t, w2t, b0, b1, b2, e1, e2, be, wpt, bp,
            block_n):
    n = kx.shape[0]
    grid = (n // block_n,)
    data_spec = pl.BlockSpec((block_n, NJ), lambda i: (i, 0))

    def rep(shape):
        return pl.BlockSpec(shape, lambda i: tuple(0 for _ in shape))

    return pl.pallas_call(
        _body,
        grid=grid,
        in_specs=[
            data_spec, data_spec, data_spec,
            rep((3, 64)), rep((3, 64)), rep((3, 64)),
            rep((1, 64)), rep((1, 64)), rep((1, 64)),
            rep((64, 64)), rep((64, 64)), rep((1, 64)),
            rep((384, 128)), rep((1, 128)),
        ],
        out_specs=pl.BlockSpec((block_n, 128), lambda i: (i, 0)),
        out_shape=jax.ShapeDtypeStruct((n, 128), jnp.float32),
    )(kx, ky, sc, w0t, w1t, w2t, b0, b1, b2, e1, e2, be, wpt, bp)


def kernel(keypoints, scores, W0, b0, W1, b1, W2, b2, We, be, Wp, bp):
    kx = keypoints[:, :, 0]
    ky = keypoints[:, :, 1]
    return _encode(
        kx, ky, scores,
        W0.T, W1.T, W2.T,
        b0[None, :], b1[None, :], b2[None, :],
        We[:, :64].T, We[:, 64:].T, be[None, :],
        Wp.T, bp[None, :],
        block_n=256)
